# baseline (speedup 1.0000x reference)
import jax
import jax.numpy as jnp
from jax import lax
from jax.experimental import pallas as pl
from jax.experimental.pallas import tpu as pltpu
from jax.experimental.pallas import tpu_sc as plsc

NC, NS, L = 2, 16, 16

N = 10000
NP = 10240
E = 320000
H = 128
C = 121
HV = H // L

NPT = NP // NS
RCH = 128
NRC = NPT // RCH

EPT = E // NS
ECH = 100
SLB = 2000
NSL = EPT // SLB
NCS = SLB // ECH
NBUF = 3
EU = 10
RC2 = 80
NR2 = NPT // RC2

_MESH = dict(core_axis_name="c", subcore_axis_name="s", num_cores=NC,
             num_subcores=NS)


def _emb_body(emb_hbm, fi_hbm, psw_hbm, bias_hbm, h_hbm,
              fi_v, psw_v, bias_v, rows_v, sem):
    c = lax.axis_index("c")
    s = lax.axis_index("s")

    @pl.when(c == 0)
    def _():
        pltpu.sync_copy(fi_hbm.at[s], fi_v)
        pltpu.sync_copy(psw_hbm.at[s], psw_v)
        pltpu.sync_copy(bias_hbm, bias_v)

        def chunk(k, carry):
            pltpu.async_copy(emb_hbm.at[fi_v.at[k]], rows_v, sem).wait()

            def row(r, carry2):
                wv = plsc.load_gather(
                    psw_v, [jnp.full((L,), k * RCH, jnp.int32) + r])
                for j in range(HV):
                    v = rows_v[r, pl.ds(j * L, L)]
                    b = bias_v[pl.ds(j * L, L)]
                    rows_v[r, pl.ds(j * L, L)] = jnp.maximum(v * wv + b, 0.0)
                return carry2

            lax.fori_loop(0, RCH, row, 0)
            base = (s * NRC + k) * RCH
            pltpu.sync_copy(rows_v, h_hbm.at[pl.ds(base, RCH)])
            return carry

        lax.fori_loop(0, NRC, chunk, 0)


_emb_call = pl.kernel(
    _emb_body,
    out_type=jax.ShapeDtypeStruct((NP, H), jnp.float32),
    mesh=plsc.VectorSubcoreMesh(**_MESH),
    compiler_params=pltpu.CompilerParams(needs_layout_passes=False),
    scratch_types=[
        pltpu.VMEM((NRC, RCH), jnp.int32),
        pltpu.VMEM((NPT,), jnp.float32),
        pltpu.VMEM((H,), jnp.float32),
        pltpu.VMEM((RCH, H), jnp.float32),
        pltpu.SemaphoreType.DMA,
    ],
)


def _edge_body(h_hbm, src_hbm, dst_hbm, w_hbm, zer_hbm, out_hbm,
               src_v, dst_v, w_v, rows0, rows1, rows2, acc,
               g0, g1, g2, s0, s1, s2):
    c = lax.axis_index("c")
    s = lax.axis_index("s")
    rows = (rows0, rows1, rows2)
    gsem = (g0, g1, g2)
    ssem = (s0, s1, s2)

    pltpu.sync_copy(zer_hbm, rows0.at[pl.ds(0, RC2)])
    for k in range(NR2):
        pltpu.sync_copy(rows0.at[pl.ds(0, RC2)],
                        acc.at[pl.ds((s * NR2 + k) * RC2, RC2)])
    plsc.subcore_barrier()

    def scale(rows_v, base_e):
        bvec = jnp.full((L,), base_e, jnp.int32)

        @plsc.parallel_loop(0, ECH, 1, unroll=EU)
        def _(e):
            wv = plsc.load_gather(w_v, [bvec + e])
            for j in range(HV):
                rows_v[e, pl.ds(j * L, L)] = rows_v[e, pl.ds(j * L, L)] * wv

    def slab(t, carry0):
        pltpu.sync_copy(src_hbm.at[s, t], src_v)
        pltpu.sync_copy(dst_hbm.at[s, t], dst_v)
        pltpu.sync_copy(w_hbm.at[c, s, t], w_v)

        pltpu.async_copy(h_hbm.at[src_v.at[0]], rows[0], gsem[0])
        pltpu.async_copy(h_hbm.at[src_v.at[1]], rows[1], gsem[1])
        pending = [False] * NBUF
        for i in range(NCS):
            b = i % NBUF
            pltpu.make_async_copy(h_hbm.at[src_v.at[i]], rows[b],
                                  gsem[b]).wait()
            if i + 2 < NCS:
                b2 = (i + 2) % NBUF
                if pending[b2]:
                    pltpu.make_async_copy(
                        rows[b2], acc.at[dst_v.at[0]], ssem[b2]).wait()
                    pending[b2] = False
                pltpu.async_copy(h_hbm.at[src_v.at[i + 2]], rows[b2],
                                 gsem[b2])
            scale(rows[b], i * ECH)
            pltpu.async_copy(rows[b], acc.at[dst_v.at[i]], ssem[b], add=True)
            pending[b] = True
        for b in range(NBUF):
            if pending[b]:
                pltpu.make_async_copy(rows[b], acc.at[dst_v.at[0]],
                                      ssem[b]).wait()
        return carry0

    lax.fori_loop(0, NSL, slab, 0)

    plsc.subcore_barrier()
    for k in range(NR2):
        base = (s * NR2 + k) * RC2
        pltpu.sync_copy(acc.at[pl.ds(base, RC2)], rows0.at[pl.ds(0, RC2)])
        pltpu.sync_copy(rows0.at[pl.ds(0, RC2)], out_hbm.at[c, pl.ds(base, RC2)])


_edge_call = pl.kernel(
    _edge_body,
    out_type=jax.ShapeDtypeStruct((NC, NP, H), jnp.float32),
    mesh=plsc.VectorSubcoreMesh(**_MESH),
    compiler_params=pltpu.CompilerParams(needs_layout_passes=False),
    scratch_types=[
        pltpu.VMEM((NCS, ECH), jnp.int32),
        pltpu.VMEM((NCS, ECH), jnp.int32),
        pltpu.VMEM((SLB,), jnp.float32),
        pltpu.VMEM((ECH, H), jnp.float32),
        pltpu.VMEM((ECH, H), jnp.float32),
        pltpu.VMEM((ECH, H), jnp.float32),
        pltpu.VMEM_SHARED((NP, H), jnp.float32),
        pltpu.SemaphoreType.DMA,
        pltpu.SemaphoreType.DMA,
        pltpu.SemaphoreType.DMA,
        pltpu.SemaphoreType.DMA,
        pltpu.SemaphoreType.DMA,
        pltpu.SemaphoreType.DMA,
    ],
)

BM = 1024


def _dense_body(p_ref, r_ref, w_ref, b_ref, o_ref):
    x = lax.dot_general(p_ref[...], w_ref[...], (((1,), (1,)), ((), ())),
                        preferred_element_type=jnp.float32)
    o_ref[...] = jnp.maximum(x + b_ref[...], 0.0) + r_ref[...]


_dense_call = pl.pallas_call(
    _dense_body,
    grid=(NP // BM,),
    in_specs=[
        pl.BlockSpec((BM, H), lambda i: (i, 0)),
        pl.BlockSpec((BM, H), lambda i: (i, 0)),
        pl.BlockSpec((H, H), lambda i: (0, 0)),
        pl.BlockSpec((1, H), lambda i: (0, 0)),
    ],
    out_specs=pl.BlockSpec((BM, H), lambda i: (i, 0)),
    out_shape=jax.ShapeDtypeStruct((NP, H), jnp.float32),
)


def _final_body(p_ref, r_ref, w_ref, b_ref, wo_ref, bo_ref, o_ref):
    x = lax.dot_general(p_ref[...], w_ref[...], (((1,), (1,)), ((), ())),
                        preferred_element_type=jnp.float32)
    x = jnp.maximum(x + b_ref[...], 0.0) + r_ref[...]
    o_ref[...] = lax.dot_general(x, wo_ref[...], (((1,), (1,)), ((), ())),
                                 preferred_element_type=jnp.float32) + bo_ref[...]


_final_call = pl.pallas_call(
    _final_body,
    grid=(NP // BM,),
    in_specs=[
        pl.BlockSpec((BM, H), lambda i: (i, 0)),
        pl.BlockSpec((BM, H), lambda i: (i, 0)),
        pl.BlockSpec((H, H), lambda i: (0, 0)),
        pl.BlockSpec((1, H), lambda i: (0, 0)),
        pl.BlockSpec((H, H), lambda i: (0, 0)),
        pl.BlockSpec((1, H), lambda i: (0, 0)),
    ],
    out_specs=pl.BlockSpec((BM, H), lambda i: (i, 0)),
    out_shape=jax.ShapeDtypeStruct((NP, H), jnp.float32),
)


def kernel(feat_idx, offsets, per_sample_weights, edge_index, w_ppi, w_self,
           emb_table, input_bias, W1, b1, W2, b2, Wout, bout):
    del offsets

    pad = NP - N
    fi = jnp.concatenate([feat_idx.astype(jnp.int32),
                          jnp.zeros((pad,), jnp.int32)]).reshape(NS, NRC, RCH)
    psw = jnp.concatenate([per_sample_weights,
                           jnp.zeros((pad,), jnp.float32)]).reshape(NS, NPT)
    h = _emb_call(emb_table, fi, psw, input_bias)

    src = edge_index[0].astype(jnp.int32).reshape(NS, NSL, NCS, ECH)
    dst = edge_index[1].astype(jnp.int32).reshape(NS, NSL, NCS, ECH)
    wst = jnp.stack([w_ppi, w_self]).reshape(NC, NS, NSL, SLB)
    zer = jnp.zeros((RC2, H), jnp.float32)

    pr = _edge_call(h, src, dst, wst, zer)
    h = _dense_call(pr[0], pr[1], W1, b1.reshape(1, H))

    pr = _edge_call(h, src, dst, wst, zer)
    wout_p = jnp.zeros((H, H), jnp.float32).at[:C].set(Wout)
    bout_p = jnp.zeros((1, H), jnp.float32).at[0, :C].set(bout)
    out = _final_call(pr[0], pr[1], W2, b2.reshape(1, H), wout_p, bout_p)
    return out[:N, :C]


# speedup vs baseline: 1.0306x; 1.0306x over previous
import jax
import jax.numpy as jnp
from jax import lax
from jax.experimental import pallas as pl
from jax.experimental.pallas import tpu as pltpu
from jax.experimental.pallas import tpu_sc as plsc

NC, NS, L = 2, 16, 16

N = 10000
NP = 10240
E = 320000
H = 128
C = 121
HV = H // L

NPT = NP // NS
RCH = 128
NRC = NPT // RCH

EPT = E // NS
ECH = 100
SLB = 2000
NSL = EPT // SLB
NCS = SLB // ECH
NBUF = 3
EU = 5
RC2 = 80
NR2 = NPT // RC2

_MESH = dict(core_axis_name="c", subcore_axis_name="s", num_cores=NC,
             num_subcores=NS)


def _emb_body(emb_hbm, fi_hbm, psw_hbm, bias_hbm, h_hbm,
              fi_v, psw_v, bias_v, rows_v, sem):
    c = lax.axis_index("c")
    s = lax.axis_index("s")

    @pl.when(c == 0)
    def _():
        pltpu.sync_copy(fi_hbm.at[s], fi_v)
        pltpu.sync_copy(psw_hbm.at[s], psw_v)
        pltpu.sync_copy(bias_hbm, bias_v)

        def chunk(k, carry):
            pltpu.async_copy(emb_hbm.at[fi_v.at[k]], rows_v, sem).wait()

            def row(r, carry2):
                wv = plsc.load_gather(
                    psw_v, [jnp.full((L,), k * RCH, jnp.int32) + r])
                for j in range(HV):
                    v = rows_v[r, pl.ds(j * L, L)]
                    b = bias_v[pl.ds(j * L, L)]
                    rows_v[r, pl.ds(j * L, L)] = jnp.maximum(v * wv + b, 0.0)
                return carry2

            lax.fori_loop(0, RCH, row, 0)
            base = (s * NRC + k) * RCH
            pltpu.sync_copy(rows_v, h_hbm.at[pl.ds(base, RCH)])
            return carry

        lax.fori_loop(0, NRC, chunk, 0)


_emb_call = pl.kernel(
    _emb_body,
    out_type=jax.ShapeDtypeStruct((NP, H), jnp.float32),
    mesh=plsc.VectorSubcoreMesh(**_MESH),
    compiler_params=pltpu.CompilerParams(needs_layout_passes=False),
    scratch_types=[
        pltpu.VMEM((NRC, RCH), jnp.int32),
        pltpu.VMEM((NPT,), jnp.float32),
        pltpu.VMEM((H,), jnp.float32),
        pltpu.VMEM((RCH, H), jnp.float32),
        pltpu.SemaphoreType.DMA,
    ],
)


def _edge_body(h_hbm, src_hbm, dst_hbm, w_hbm, zer_hbm, out_hbm,
               src_v, dst_v, w_v, rows0, rows1, rows2, acc,
               g0, g1, g2, s0, s1, s2):
    c = lax.axis_index("c")
    s = lax.axis_index("s")
    rows = (rows0, rows1, rows2)
    gsem = (g0, g1, g2)
    ssem = (s0, s1, s2)

    pltpu.sync_copy(zer_hbm, rows0.at[pl.ds(0, RC2)])
    for k in range(NR2):
        pltpu.sync_copy(rows0.at[pl.ds(0, RC2)],
                        acc.at[pl.ds((s * NR2 + k) * RC2, RC2)])
    plsc.subcore_barrier()

    def scale(rows_v, base_e):
        bvec = jnp.full((L,), base_e, jnp.int32)

        @plsc.parallel_loop(0, ECH, 1, unroll=EU)
        def _(e):
            wv = plsc.load_gather(w_v, [bvec + e])
            for j in range(HV):
                rows_v[e, pl.ds(j * L, L)] = rows_v[e, pl.ds(j * L, L)] * wv

    def slab(t, carry0):
        pltpu.sync_copy(src_hbm.at[s, t], src_v)
        pltpu.sync_copy(dst_hbm.at[s, t], dst_v)
        pltpu.sync_copy(w_hbm.at[c, s, t], w_v)

        pltpu.async_copy(h_hbm.at[src_v.at[0]], rows[0], gsem[0])
        pltpu.async_copy(h_hbm.at[src_v.at[1]], rows[1], gsem[1])
        pending = [False] * NBUF
        for i in range(NCS):
            b = i % NBUF
            pltpu.make_async_copy(h_hbm.at[src_v.at[i]], rows[b],
                                  gsem[b]).wait()
            if i + 2 < NCS:
                b2 = (i + 2) % NBUF
                if pending[b2]:
                    pltpu.make_async_copy(
                        rows[b2], acc.at[dst_v.at[0]], ssem[b2]).wait()
                    pending[b2] = False
                pltpu.async_copy(h_hbm.at[src_v.at[i + 2]], rows[b2],
                                 gsem[b2])
            scale(rows[b], i * ECH)
            pltpu.async_copy(rows[b], acc.at[dst_v.at[i]], ssem[b], add=True)
            pending[b] = True
        for b in range(NBUF):
            if pending[b]:
                pltpu.make_async_copy(rows[b], acc.at[dst_v.at[0]],
                                      ssem[b]).wait()
        return carry0

    lax.fori_loop(0, NSL, slab, 0)

    plsc.subcore_barrier()
    for k in range(NR2):
        base = (s * NR2 + k) * RC2
        pltpu.sync_copy(acc.at[pl.ds(base, RC2)], rows0.at[pl.ds(0, RC2)])
        pltpu.sync_copy(rows0.at[pl.ds(0, RC2)], out_hbm.at[c, pl.ds(base, RC2)])


_edge_call = pl.kernel(
    _edge_body,
    out_type=jax.ShapeDtypeStruct((NC, NP, H), jnp.float32),
    mesh=plsc.VectorSubcoreMesh(**_MESH),
    compiler_params=pltpu.CompilerParams(needs_layout_passes=False),
    scratch_types=[
        pltpu.VMEM((NCS, ECH), jnp.int32),
        pltpu.VMEM((NCS, ECH), jnp.int32),
        pltpu.VMEM((SLB,), jnp.float32),
        pltpu.VMEM((ECH, H), jnp.float32),
        pltpu.VMEM((ECH, H), jnp.float32),
        pltpu.VMEM((ECH, H), jnp.float32),
        pltpu.VMEM_SHARED((NP, H), jnp.float32),
        pltpu.SemaphoreType.DMA,
        pltpu.SemaphoreType.DMA,
        pltpu.SemaphoreType.DMA,
        pltpu.SemaphoreType.DMA,
        pltpu.SemaphoreType.DMA,
        pltpu.SemaphoreType.DMA,
    ],
)

BM = 1024


def _dense_body(p_ref, r_ref, w_ref, b_ref, o_ref):
    x = lax.dot_general(p_ref[...], w_ref[...], (((1,), (1,)), ((), ())),
                        preferred_element_type=jnp.float32)
    o_ref[...] = jnp.maximum(x + b_ref[...], 0.0) + r_ref[...]


_dense_call = pl.pallas_call(
    _dense_body,
    grid=(NP // BM,),
    in_specs=[
        pl.BlockSpec((BM, H), lambda i: (i, 0)),
        pl.BlockSpec((BM, H), lambda i: (i, 0)),
        pl.BlockSpec((H, H), lambda i: (0, 0)),
        pl.BlockSpec((1, H), lambda i: (0, 0)),
    ],
    out_specs=pl.BlockSpec((BM, H), lambda i: (i, 0)),
    out_shape=jax.ShapeDtypeStruct((NP, H), jnp.float32),
)


def _final_body(p_ref, r_ref, w_ref, b_ref, wo_ref, bo_ref, o_ref):
    x = lax.dot_general(p_ref[...], w_ref[...], (((1,), (1,)), ((), ())),
                        preferred_element_type=jnp.float32)
    x = jnp.maximum(x + b_ref[...], 0.0) + r_ref[...]
    o_ref[...] = lax.dot_general(x, wo_ref[...], (((1,), (1,)), ((), ())),
                                 preferred_element_type=jnp.float32) + bo_ref[...]


_final_call = pl.pallas_call(
    _final_body,
    grid=(NP // BM,),
    in_specs=[
        pl.BlockSpec((BM, H), lambda i: (i, 0)),
        pl.BlockSpec((BM, H), lambda i: (i, 0)),
        pl.BlockSpec((H, H), lambda i: (0, 0)),
        pl.BlockSpec((1, H), lambda i: (0, 0)),
        pl.BlockSpec((H, H), lambda i: (0, 0)),
        pl.BlockSpec((1, H), lambda i: (0, 0)),
    ],
    out_specs=pl.BlockSpec((BM, H), lambda i: (i, 0)),
    out_shape=jax.ShapeDtypeStruct((NP, H), jnp.float32),
)


def kernel(feat_idx, offsets, per_sample_weights, edge_index, w_ppi, w_self,
           emb_table, input_bias, W1, b1, W2, b2, Wout, bout):
    del offsets

    pad = NP - N
    fi = jnp.concatenate([feat_idx.astype(jnp.int32),
                          jnp.zeros((pad,), jnp.int32)]).reshape(NS, NRC, RCH)
    psw = jnp.concatenate([per_sample_weights,
                           jnp.zeros((pad,), jnp.float32)]).reshape(NS, NPT)
    h = _emb_call(emb_table, fi, psw, input_bias)

    src = edge_index[0].astype(jnp.int32).reshape(NS, NSL, NCS, ECH)
    dst = edge_index[1].astype(jnp.int32).reshape(NS, NSL, NCS, ECH)
    wst = jnp.stack([w_ppi, w_self]).reshape(NC, NS, NSL, SLB)
    zer = jnp.zeros((RC2, H), jnp.float32)

    pr = _edge_call(h, src, dst, wst, zer)
    h = _dense_call(pr[0], pr[1], W1, b1.reshape(1, H))

    pr = _edge_call(h, src, dst, wst, zer)
    wout_p = jnp.zeros((H, H), jnp.float32).at[:C].set(Wout)
    bout_p = jnp.zeros((1, H), jnp.float32).at[0, :C].set(bout)
    out = _final_call(pr[0], pr[1], W2, b2.reshape(1, H), wout_p, bout_p)
    return out[:N, :C]


# speedup vs baseline: 1.0926x; 1.0602x over previous
import jax
import jax.numpy as jnp
from jax import lax
from jax.experimental import pallas as pl
from jax.experimental.pallas import tpu as pltpu
from jax.experimental.pallas import tpu_sc as plsc

NC, NS, L = 2, 16, 16

N = 10000
NP = 10240
E = 320000
H = 128
C = 121
HV = H // L

NPT = NP // NS
RCH = 128
NRC = NPT // RCH

EPT = E // NS
ECH = 125
SLB = 2000
NSL = EPT // SLB
NCS = SLB // ECH
NBUF = 2
EU = 5
RC2 = 80
NR2 = NPT // RC2

_MESH = dict(core_axis_name="c", subcore_axis_name="s", num_cores=NC,
             num_subcores=NS)


def _emb_body(emb_hbm, fi_hbm, psw_hbm, bias_hbm, h_hbm,
              fi_v, psw_v, bias_v, rows_v, sem):
    c = lax.axis_index("c")
    s = lax.axis_index("s")

    @pl.when(c == 0)
    def _():
        pltpu.sync_copy(fi_hbm.at[s], fi_v)
        pltpu.sync_copy(psw_hbm.at[s], psw_v)
        pltpu.sync_copy(bias_hbm, bias_v)

        def chunk(k, carry):
            pltpu.async_copy(emb_hbm.at[fi_v.at[k]], rows_v, sem).wait()

            def row(r, carry2):
                wv = plsc.load_gather(
                    psw_v, [jnp.full((L,), k * RCH, jnp.int32) + r])
                for j in range(HV):
                    v = rows_v[r, pl.ds(j * L, L)]
                    b = bias_v[pl.ds(j * L, L)]
                    rows_v[r, pl.ds(j * L, L)] = jnp.maximum(v * wv + b, 0.0)
                return carry2

            lax.fori_loop(0, RCH, row, 0)
            base = (s * NRC + k) * RCH
            pltpu.sync_copy(rows_v, h_hbm.at[pl.ds(base, RCH)])
            return carry

        lax.fori_loop(0, NRC, chunk, 0)


_emb_call = pl.kernel(
    _emb_body,
    out_type=jax.ShapeDtypeStruct((NP, H), jnp.float32),
    mesh=plsc.VectorSubcoreMesh(**_MESH),
    compiler_params=pltpu.CompilerParams(needs_layout_passes=False),
    scratch_types=[
        pltpu.VMEM((NRC, RCH), jnp.int32),
        pltpu.VMEM((NPT,), jnp.float32),
        pltpu.VMEM((H,), jnp.float32),
        pltpu.VMEM((RCH, H), jnp.float32),
        pltpu.SemaphoreType.DMA,
    ],
)


def _edge_body(h_hbm, src_hbm, dst_hbm, w_hbm, zer_hbm, out_hbm,
               src_v, dst_v, w_v, rows0, rows1, acc,
               g0, g1, s0, s1):
    c = lax.axis_index("c")
    s = lax.axis_index("s")
    rows = (rows0, rows1)
    gsem = (g0, g1)
    ssem = (s0, s1)

    pltpu.sync_copy(zer_hbm, rows0.at[pl.ds(0, RC2)])
    for k in range(NR2):
        pltpu.sync_copy(rows0.at[pl.ds(0, RC2)],
                        acc.at[pl.ds((s * NR2 + k) * RC2, RC2)])
    plsc.subcore_barrier()

    def scale(rows_v, base_e):
        bvec = jnp.full((L,), base_e, jnp.int32)

        @plsc.parallel_loop(0, ECH, 1, unroll=EU)
        def _(e):
            wv = plsc.load_gather(w_v, [bvec + e])
            for j in range(HV):
                rows_v[e, pl.ds(j * L, L)] = rows_v[e, pl.ds(j * L, L)] * wv

    def slab(t, carry0):
        pltpu.sync_copy(src_hbm.at[s, t], src_v)
        pltpu.sync_copy(dst_hbm.at[s, t], dst_v)
        pltpu.sync_copy(w_hbm.at[c, s, t], w_v)

        pltpu.async_copy(h_hbm.at[src_v.at[0]], rows[0], gsem[0])
        pltpu.async_copy(h_hbm.at[src_v.at[1]], rows[1], gsem[1])
        pending = [False] * NBUF
        for i in range(NCS):
            b = i % NBUF
            pltpu.make_async_copy(h_hbm.at[src_v.at[i]], rows[b],
                                  gsem[b]).wait()
            if i + 2 < NCS:
                b2 = (i + 2) % NBUF
                if pending[b2]:
                    pltpu.make_async_copy(
                        rows[b2], acc.at[dst_v.at[0]], ssem[b2]).wait()
                    pending[b2] = False
                pltpu.async_copy(h_hbm.at[src_v.at[i + 2]], rows[b2],
                                 gsem[b2])
            scale(rows[b], i * ECH)
            pltpu.async_copy(rows[b], acc.at[dst_v.at[i]], ssem[b], add=True)
            pending[b] = True
        for b in range(NBUF):
            if pending[b]:
                pltpu.make_async_copy(rows[b], acc.at[dst_v.at[0]],
                                      ssem[b]).wait()
        return carry0

    lax.fori_loop(0, NSL, slab, 0)

    plsc.subcore_barrier()
    for k in range(NR2):
        base = (s * NR2 + k) * RC2
        pltpu.sync_copy(acc.at[pl.ds(base, RC2)], rows0.at[pl.ds(0, RC2)])
        pltpu.sync_copy(rows0.at[pl.ds(0, RC2)], out_hbm.at[c, pl.ds(base, RC2)])


_edge_call = pl.kernel(
    _edge_body,
    out_type=jax.ShapeDtypeStruct((NC, NP, H), jnp.float32),
    mesh=plsc.VectorSubcoreMesh(**_MESH),
    compiler_params=pltpu.CompilerParams(needs_layout_passes=False),
    scratch_types=[
        pltpu.VMEM((NCS, ECH), jnp.int32),
        pltpu.VMEM((NCS, ECH), jnp.int32),
        pltpu.VMEM((SLB,), jnp.float32),
        pltpu.VMEM((ECH, H), jnp.float32),
        pltpu.VMEM((ECH, H), jnp.float32),
        pltpu.VMEM_SHARED((NP, H), jnp.float32),
        pltpu.SemaphoreType.DMA,
        pltpu.SemaphoreType.DMA,
        pltpu.SemaphoreType.DMA,
        pltpu.SemaphoreType.DMA,
    ],
)

BM = 1024


def _dense_body(p_ref, r_ref, w_ref, b_ref, o_ref):
    x = lax.dot_general(p_ref[...], w_ref[...], (((1,), (1,)), ((), ())),
                        preferred_element_type=jnp.float32)
    o_ref[...] = jnp.maximum(x + b_ref[...], 0.0) + r_ref[...]


_dense_call = pl.pallas_call(
    _dense_body,
    grid=(NP // BM,),
    in_specs=[
        pl.BlockSpec((BM, H), lambda i: (i, 0)),
        pl.BlockSpec((BM, H), lambda i: (i, 0)),
        pl.BlockSpec((H, H), lambda i: (0, 0)),
        pl.BlockSpec((1, H), lambda i: (0, 0)),
    ],
    out_specs=pl.BlockSpec((BM, H), lambda i: (i, 0)),
    out_shape=jax.ShapeDtypeStruct((NP, H), jnp.float32),
)


def _final_body(p_ref, r_ref, w_ref, b_ref, wo_ref, bo_ref, o_ref):
    x = lax.dot_general(p_ref[...], w_ref[...], (((1,), (1,)), ((), ())),
                        preferred_element_type=jnp.float32)
    x = jnp.maximum(x + b_ref[...], 0.0) + r_ref[...]
    o_ref[...] = lax.dot_general(x, wo_ref[...], (((1,), (1,)), ((), ())),
                                 preferred_element_type=jnp.float32) + bo_ref[...]


_final_call = pl.pallas_call(
    _final_body,
    grid=(NP // BM,),
    in_specs=[
        pl.BlockSpec((BM, H), lambda i: (i, 0)),
        pl.BlockSpec((BM, H), lambda i: (i, 0)),
        pl.BlockSpec((H, H), lambda i: (0, 0)),
        pl.BlockSpec((1, H), lambda i: (0, 0)),
        pl.BlockSpec((H, H), lambda i: (0, 0)),
        pl.BlockSpec((1, H), lambda i: (0, 0)),
    ],
    out_specs=pl.BlockSpec((BM, H), lambda i: (i, 0)),
    out_shape=jax.ShapeDtypeStruct((NP, H), jnp.float32),
)


def kernel(feat_idx, offsets, per_sample_weights, edge_index, w_ppi, w_self,
           emb_table, input_bias, W1, b1, W2, b2, Wout, bout):
    del offsets

    pad = NP - N
    fi = jnp.concatenate([feat_idx.astype(jnp.int32),
                          jnp.zeros((pad,), jnp.int32)]).reshape(NS, NRC, RCH)
    psw = jnp.concatenate([per_sample_weights,
                           jnp.zeros((pad,), jnp.float32)]).reshape(NS, NPT)
    h = _emb_call(emb_table, fi, psw, input_bias)

    src = edge_index[0].astype(jnp.int32).reshape(NS, NSL, NCS, ECH)
    dst = edge_index[1].astype(jnp.int32).reshape(NS, NSL, NCS, ECH)
    wst = jnp.stack([w_ppi, w_self]).reshape(NC, NS, NSL, SLB)
    zer = jnp.zeros((RC2, H), jnp.float32)

    pr = _edge_call(h, src, dst, wst, zer)
    h = _dense_call(pr[0], pr[1], W1, b1.reshape(1, H))

    pr = _edge_call(h, src, dst, wst, zer)
    wout_p = jnp.zeros((H, H), jnp.float32).at[:C].set(Wout)
    bout_p = jnp.zeros((1, H), jnp.float32).at[0, :C].set(bout)
    out = _final_call(pr[0], pr[1], W2, b2.reshape(1, H), wout_p, bout_p)
    return out[:N, :C]
